# Initial kernel scaffold; baseline (speedup 1.0000x reference)
import jax
import jax.numpy as jnp
from jax.experimental import pallas as pl


def kernel(x_num, Wg, bg, W1, b1, W2, b2):
    raise NotImplementedError("write your pallas kernel here")


# speedup vs baseline: 2.5855x; 2.5855x over previous
import functools

import jax
import jax.numpy as jnp
from jax.experimental import pallas as pl
from jax.experimental.pallas import tpu as pltpu

B = 4096
D_IN = 256
D_BLOCK = 512
N_EXPERTS = 8
TOP_K = 2
D_OUT = 10

BT = 512


def _fused_moe_body(x_ref, wg_ref, bg_ref, w1_ref, b1_ref, w2_ref, b2_ref,
                    out_ref):
    x = x_ref[...]
    logits = jnp.dot(x, wg_ref[...], preferred_element_type=jnp.float32)
    logits = logits + bg_ref[...]

    eids = jax.lax.broadcasted_iota(jnp.int32, logits.shape, 1)
    m1 = jnp.max(logits, axis=-1, keepdims=True)
    i1 = jnp.min(jnp.where(logits == m1, eids, N_EXPERTS), axis=-1,
                 keepdims=True)
    neg = jnp.float32(-jnp.inf)
    masked = jnp.where(eids == i1, neg, logits)
    m2 = jnp.max(masked, axis=-1, keepdims=True)
    i2 = jnp.min(jnp.where(masked == m2, eids, N_EXPERTS), axis=-1,
                 keepdims=True)
    e2 = jnp.exp(m2 - m1)
    g1 = 1.0 / (1.0 + e2)
    g2 = e2 / (1.0 + e2)
    w = g1 * (eids == i1).astype(jnp.float32) + g2 * (eids == i2).astype(
        jnp.float32)

    acc = jnp.zeros((x.shape[0], D_OUT), dtype=jnp.float32)
    for e in range(N_EXPERTS):
        h = jnp.dot(x, w1_ref[e], preferred_element_type=jnp.float32)
        h = jnp.maximum(h + b1_ref[e:e + 1, :], 0.0)
        o = jnp.dot(h, w2_ref[e], preferred_element_type=jnp.float32)
        o = o + b2_ref[e:e + 1, :]
        acc = acc + w[:, e:e + 1] * o
    out_ref[...] = acc


@functools.partial(jax.jit, static_argnames=("interpret",))
def _fused_moe(x, Wg, bg2d, W1, b1, W2, b2, interpret=False):
    grid = (B // BT,)
    return pl.pallas_call(
        _fused_moe_body,
        grid=grid,
        in_specs=[
            pl.BlockSpec((BT, D_IN), lambda i: (i, 0)),
            pl.BlockSpec((D_IN, N_EXPERTS), lambda i: (0, 0)),
            pl.BlockSpec((1, N_EXPERTS), lambda i: (0, 0)),
            pl.BlockSpec((N_EXPERTS, D_IN, D_BLOCK), lambda i: (0, 0, 0)),
            pl.BlockSpec((N_EXPERTS, D_BLOCK), lambda i: (0, 0)),
            pl.BlockSpec((N_EXPERTS, D_BLOCK, D_OUT), lambda i: (0, 0, 0)),
            pl.BlockSpec((N_EXPERTS, D_OUT), lambda i: (0, 0)),
        ],
        out_specs=pl.BlockSpec((BT, D_OUT), lambda i: (i, 0)),
        out_shape=jax.ShapeDtypeStruct((B, D_OUT), jnp.float32),
        interpret=interpret,
    )(x, Wg, bg2d, W1, b1, W2, b2)


def kernel(x_num, Wg, bg, W1, b1, W2, b2):
    x = jnp.reshape(x_num, (x_num.shape[0], -1))
    return _fused_moe(x, Wg, bg.reshape(1, N_EXPERTS), W1, b1, W2, b2)
